# Initial kernel scaffold; baseline (speedup 1.0000x reference)
import jax
import jax.numpy as jnp
from jax.experimental import pallas as pl


def kernel(x, edge_index, batch, W1, b1, W2, b2, W3, b3, W4, b4, W5, b5, Wlin, blin):
    raise NotImplementedError("write your pallas kernel here")


# speedup vs baseline: 8.3971x; 8.3971x over previous
import functools

import jax
import jax.numpy as jnp
from jax import lax
from jax.experimental import pallas as pl
from jax.experimental.pallas import tpu as pltpu
from jax.experimental.pallas import tpu_sc as plsc

NN = 10000
EE = 320000
HH = 128
GG = 64

NCORE = 2
NSUB = 16
NWORK = NCORE * NSUB
CHUNK = 128
EPW = (EE + NWORK - 1) // NWORK
NCHUNK = (EPW + CHUNK - 1) // CHUNK
EPAD = NCHUNK * CHUNK * NWORK
NPAD = 10112
RPT = NPAD // NSUB

BR = 32
NBLK = NPAD // BR
GP = 72


def _deg_sc_body(dst3, ones_h, zeros_h, pdeg, didx, ones_v, acc):
    c = lax.axis_index("c")
    s = lax.axis_index("s")
    w = s * NCORE + c
    r0 = s * RPT
    pltpu.sync_copy(zeros_h.at[pl.ds(r0, RPT)], acc.at[pl.ds(r0, RPT)])
    pltpu.sync_copy(ones_h, ones_v)
    pltpu.sync_copy(dst3.at[w], didx)
    plsc.subcore_barrier()

    def body(j, carry):
        pltpu.sync_copy(ones_v, acc.at[didx.at[j]], add=True)
        return carry

    lax.fori_loop(0, NCHUNK, body, 0)
    plsc.subcore_barrier()
    pltpu.sync_copy(acc.at[pl.ds(r0, RPT)], pdeg.at[c, pl.ds(r0, RPT)])


def _edge_pass_sc_body(y, src3, dst3, zeros_h, pout, sidx, didx, gbuf, acc,
                       sem):
    c = lax.axis_index("c")
    s = lax.axis_index("s")
    w = s * NCORE + c
    r0 = s * RPT

    @pl.when(c == 0)
    def _():
        pltpu.sync_copy(y.at[pl.ds(r0, RPT)], acc.at[pl.ds(r0, RPT)])

    @pl.when(c != 0)
    def _():
        pltpu.sync_copy(zeros_h.at[pl.ds(r0, RPT)], acc.at[pl.ds(r0, RPT)])

    pltpu.sync_copy(src3.at[w], sidx)
    pltpu.sync_copy(dst3.at[w], didx)
    plsc.subcore_barrier()

    def body(j, carry):
        pltpu.async_copy(y.at[sidx.at[j]], gbuf, sem).wait()
        pltpu.sync_copy(gbuf, acc.at[didx.at[j]], add=True)
        return carry

    lax.fori_loop(0, NCHUNK, body, 0)
    plsc.subcore_barrier()
    pltpu.sync_copy(acc.at[pl.ds(r0, RPT)], pout.at[c, pl.ds(r0, RPT)])


@functools.lru_cache(maxsize=None)
def _sc_mesh():
    return plsc.VectorSubcoreMesh(
        core_axis_name="c", subcore_axis_name="s",
        num_cores=NCORE, num_subcores=NSUB)


@functools.lru_cache(maxsize=None)
def _deg_sc():
    return pl.kernel(
        _deg_sc_body,
        out_type=jax.ShapeDtypeStruct((NCORE, NPAD, HH), jnp.float32),
        mesh=_sc_mesh(),
        scratch_types=[
            pltpu.VMEM((NCHUNK, CHUNK), jnp.int32),
            pltpu.VMEM((CHUNK, HH), jnp.float32),
            pltpu.VMEM_SHARED((NPAD, HH), jnp.float32),
        ],
    )


@functools.lru_cache(maxsize=None)
def _edge_pass_sc():
    return pl.kernel(
        _edge_pass_sc_body,
        out_type=jax.ShapeDtypeStruct((NCORE, NPAD, HH), jnp.float32),
        mesh=_sc_mesh(),
        scratch_types=[
            pltpu.VMEM((NCHUNK, CHUNK), jnp.int32),
            pltpu.VMEM((NCHUNK, CHUNK), jnp.int32),
            pltpu.VMEM((CHUNK, HH), jnp.float32),
            pltpu.VMEM_SHARED((NPAD, HH), jnp.float32),
            pltpu.SemaphoreType.DMA,
        ],
    )


def _run_deg(dst3, ones_h, zeros_h):
    return _deg_sc()(dst3, ones_h, zeros_h)


def _run_edge_pass(y, src3, dst3, zeros_h):
    return _edge_pass_sc()(y, src3, dst3, zeros_h)


def _prep_body(pdeg, dinv):
    deg = pdeg[0][:, 0:1] + pdeg[1][:, 0:1] + 1.0
    dinv[...] = jnp.broadcast_to(
        lax.rsqrt(jnp.maximum(deg, 1.0)), (NPAD, HH))


_prep_tc = pl.pallas_call(
    _prep_body,
    out_shape=jax.ShapeDtypeStruct((NPAD, HH), jnp.float32),
)


def _first_body(x, dinv, w1, y):
    y[...] = dinv[...] * jnp.dot(x[...], w1[...],
                                 preferred_element_type=jnp.float32)


_first_tc = pl.pallas_call(
    _first_body,
    out_shape=jax.ShapeDtypeStruct((NPAD, HH), jnp.float32),
)


def _layer_body(p, dinv, b, w, y):
    dv = dinv[...]
    h = jnp.maximum(dv * (p[0] + p[1]) + b[...], 0.0)
    y[...] = dv * jnp.dot(h, w[...], preferred_element_type=jnp.float32)


_layer_tc = pl.pallas_call(
    _layer_body,
    out_shape=jax.ShapeDtypeStruct((NPAD, HH), jnp.float32),
)


def _pool_body(bmm, p0, p1, dinv, b5, bvec, wlin, blin, out, acc):
    i = pl.program_id(0)

    @pl.when(i == 0)
    def _():
        acc[...] = jnp.full((GP, HH), -jnp.inf, jnp.float32)

    h = jnp.maximum(dinv[...] * (p0[...] + p1[...]) + b5[...], 0.0)
    bv = bvec[...]
    lo = bmm[0, 0, 0]
    hi = bmm[0, 0, 1]

    def body(g, carry):
        m = jnp.where(bv == g, h, -jnp.inf)
        mx = jnp.max(m, axis=0, keepdims=True)
        acc[pl.ds(g, 1), :] = jnp.maximum(acc[pl.ds(g, 1), :], mx)
        return carry

    lax.fori_loop(lo, hi + 1, body, 0)

    @pl.when(i == NBLK - 1)
    def _():
        out[...] = (jnp.dot(acc[0:GG, :], wlin[...],
                            preferred_element_type=jnp.float32) + blin[...])


def _make_pool(ncls):
    return pl.pallas_call(
        _pool_body,
        grid=(NBLK,),
        in_specs=[
            pl.BlockSpec((1, 1, 2), lambda i: (i, 0, 0), memory_space=pltpu.SMEM),
            pl.BlockSpec((BR, HH), lambda i: (i, 0)),
            pl.BlockSpec((BR, HH), lambda i: (i, 0)),
            pl.BlockSpec((BR, HH), lambda i: (i, 0)),
            pl.BlockSpec((1, HH), lambda i: (0, 0)),
            pl.BlockSpec((BR, 1), lambda i: (i, 0)),
            pl.BlockSpec((HH, ncls), lambda i: (0, 0)),
            pl.BlockSpec((1, ncls), lambda i: (0, 0)),
        ],
        out_specs=pl.BlockSpec((GG, ncls), lambda i: (0, 0)),
        out_shape=jax.ShapeDtypeStruct((GG, ncls), jnp.float32),
        scratch_shapes=[pltpu.VMEM((GP, HH), jnp.float32)],
    )


@jax.jit
def kernel(x, edge_index, batch, W1, b1, W2, b2, W3, b3, W4, b4, W5, b5,
           Wlin, blin):
    ncls = Wlin.shape[1]

    src = edge_index[0].astype(jnp.int32)
    dst = edge_index[1].astype(jnp.int32)
    pad_e = EPAD - EE
    src3 = jnp.concatenate(
        [src, jnp.zeros((pad_e,), jnp.int32)]).reshape(NWORK, NCHUNK, CHUNK)
    dst3 = jnp.concatenate(
        [dst, jnp.full((pad_e,), NN, jnp.int32)]).reshape(NWORK, NCHUNK, CHUNK)

    xp = jnp.concatenate(
        [x, jnp.zeros((NPAD - NN, HH), jnp.float32)], axis=0)
    bp = jnp.concatenate(
        [batch.astype(jnp.int32), jnp.full((NPAD - NN,), GG, jnp.int32)])
    bvec = bp[:, None]
    b2d = bp.reshape(NBLK, BR)
    bmm = jnp.stack([b2d[:, 0], b2d[:, -1]], axis=1).reshape(NBLK, 1, 2)

    ones_h = jnp.ones((CHUNK, HH), jnp.float32)
    zeros_h = jnp.zeros((NPAD, HH), jnp.float32)

    biases = [b.reshape(1, HH) for b in (b1, b2, b3, b4, b5)]
    blin2 = blin.reshape(1, ncls)

    pdeg = _run_deg(dst3, ones_h, zeros_h)
    dinv = _prep_tc(pdeg)

    y = _first_tc(xp, dinv, W1)
    p = _run_edge_pass(y, src3, dst3, zeros_h)
    for W, b in ((W2, biases[0]), (W3, biases[1]), (W4, biases[2]),
                 (W5, biases[3])):
        y = _layer_tc(p, dinv, b, W)
        p = _run_edge_pass(y, src3, dst3, zeros_h)

    return _make_pool(ncls)(bmm, p[0], p[1], dinv, biases[4], bvec,
                            Wlin, blin2)
